# Initial kernel scaffold; baseline (speedup 1.0000x reference)
import jax
import jax.numpy as jnp
from jax.experimental import pallas as pl


def kernel(token_ids, weight):
    raise NotImplementedError("write your pallas kernel here")


# speedup vs baseline: 1.1107x; 1.1107x over previous
import functools

import jax
import jax.numpy as jnp
from jax import lax
from jax.experimental import pallas as pl
from jax.experimental.pallas import tpu as pltpu
from jax.experimental.pallas import tpu_sc as plsc

NC = 2
NS = 16
NW = NC * NS

B = 16384 * 50
D = 32
BPW = B // NW
C = 1280
NCHUNK = BPW // C

_mesh = plsc.VectorSubcoreMesh(core_axis_name="c", subcore_axis_name="s")


@functools.partial(
    pl.kernel,
    mesh=_mesh,
    compiler_params=pltpu.CompilerParams(use_tc_tiling_on_sc=False),
    out_type=jax.ShapeDtypeStruct((B, D), jnp.float32),
    scratch_types=[
        pltpu.VMEM((BPW,), jnp.int32),
        pltpu.VMEM((2, C, D), jnp.float32),
        pltpu.SemaphoreType.DMA,
        pltpu.SemaphoreType.DMA,
    ],
)
def _gather_kernel(idx_hbm, table_hbm, out_hbm, idx_v, rows_v, gsem, ssem):
    wid = lax.axis_index("s") * NC + lax.axis_index("c")
    base = wid * BPW
    pltpu.sync_copy(idx_hbm.at[pl.ds(base, BPW)], idx_v)

    def gather(i):
        return pltpu.async_copy(
            table_hbm.at[idx_v.at[pl.ds(i * C, C)]], rows_v.at[i % 2], gsem)

    def store(i):
        return pltpu.async_copy(
            rows_v.at[i % 2], out_hbm.at[pl.ds(base + i * C, C)], ssem)

    g = [None] * NCHUNK
    s = [None] * NCHUNK
    g[0] = gather(0)
    g[0].wait()
    s[0] = store(0)
    if NCHUNK > 1:
        g[1] = gather(1)
    for i in range(1, NCHUNK):
        g[i].wait()
        s[i] = store(i)
        if i + 1 < NCHUNK:
            s[i - 1].wait()
            g[i + 1] = gather(i + 1)
    if NCHUNK > 1:
        s[NCHUNK - 2].wait()
    s[NCHUNK - 1].wait()


def kernel(token_ids, weight):
    ids = token_ids.reshape(-1).astype(jnp.int32)
    out = _gather_kernel(ids, weight)
    return out.reshape(*token_ids.shape, weight.shape[1])


# speedup vs baseline: 1.5115x; 1.3609x over previous
import functools

import jax
import jax.numpy as jnp
from jax import lax
from jax.experimental import pallas as pl
from jax.experimental.pallas import tpu as pltpu
from jax.experimental.pallas import tpu_sc as plsc

NC = 2
NS = 16
NW = NC * NS

T = 16384
S = 50
D = 32
V = 1000000

TT = T // 128
TPW = TT // NW
NBLK = TPW * S

_mesh = plsc.VectorSubcoreMesh(core_axis_name="c", subcore_axis_name="s")


@functools.partial(
    pl.kernel,
    mesh=_mesh,
    compiler_params=pltpu.CompilerParams(
        use_tc_tiling_on_sc=False, needs_layout_passes=False),
    out_type=jax.ShapeDtypeStruct((S, D // 8, TT, 8, 128), jnp.float32),
    scratch_types=[
        pltpu.VMEM((TPW, S * 128), jnp.int32),
        pltpu.VMEM((2, 128, D), jnp.float32),
        pltpu.VMEM((2, D // 8, 8, 128), jnp.float32),
        pltpu.SemaphoreType.DMA,
        pltpu.SemaphoreType.DMA,
    ],
)
def _gather_kernel(ids_hbm, table_hbm, out_hbm, ids_v, rows_v, stg_v,
                   gsem, ssem):
    wid = lax.axis_index("s") * NC + lax.axis_index("c")
    gtt0 = wid * TPW
    pltpu.sync_copy(ids_hbm.at[pl.ds(gtt0, TPW)], ids_v)
    iota = lax.iota(jnp.int32, 16)

    def start_gather(it, b):
        tl = it // S
        s = it - tl * S
        pltpu.async_copy(
            table_hbm.at[ids_v.at[tl, pl.ds(s * 128, 128)]],
            rows_v.at[b], gsem)

    def wait_gather(b):
        pltpu.make_async_copy(
            table_hbm.at[ids_v.at[0, pl.ds(0, 128)]], rows_v.at[b],
            gsem).wait()

    def start_store(it, b):
        tl = it // S
        s = it - tl * S
        pltpu.async_copy(stg_v.at[b], out_hbm.at[s, :, gtt0 + tl], ssem)

    def wait_store(b):
        pltpu.make_async_copy(
            stg_v.at[b], out_hbm.at[0, :, gtt0], ssem).wait()

    def transpose_block(b):
        rows = rows_v.at[b]
        for e4 in range(D // 8):
            for e8 in range(8):
                col = jnp.full((16,), e4 * 8 + e8, jnp.int32)
                for t8 in range(8):
                    vals = plsc.load_gather(rows, [t8 * 16 + iota, col])
                    stg_v[b, e4, e8, pl.ds(t8 * 16, 16)] = vals

    start_gather(0, 0)
    start_gather(1, 1)

    def body(j, carry):
        for b in range(2):
            it = j * 2 + b
            wait_gather(b)

            @pl.when(it >= 2)
            def _():
                wait_store(b)

            transpose_block(b)
            start_store(it, b)

            @pl.when(it < NBLK - 2)
            def _():
                start_gather(it + 2, b)

        return carry

    lax.fori_loop(0, NBLK // 2, body, 0)
    wait_store(0)
    wait_store(1)


def kernel(token_ids, weight):
    ids2 = (token_ids.astype(jnp.int32).T
            .reshape(S, TT, 128).transpose(1, 0, 2).reshape(TT, S * 128))
    out5 = _gather_kernel(ids2, weight)
    return out5.transpose(2, 4, 0, 1, 3).reshape(T, S, D)


# speedup vs baseline: 2.4515x; 1.6218x over previous
import functools

import jax
import jax.numpy as jnp
from jax import lax
from jax.experimental import pallas as pl
from jax.experimental.pallas import tpu as pltpu
from jax.experimental.pallas import tpu_sc as plsc

NC = 2
NS = 16
NW = NC * NS

T = 16384
S = 50
D = 32
V = 1000000

TT = T // 128
TPW = TT // NW
NBLK = TPW * S

_mesh = plsc.VectorSubcoreMesh(core_axis_name="c", subcore_axis_name="s")


@functools.partial(
    pl.kernel,
    mesh=_mesh,
    compiler_params=pltpu.CompilerParams(
        use_tc_tiling_on_sc=False, needs_layout_passes=False),
    out_type=jax.ShapeDtypeStruct((S, D // 8, TT, 8, 128), jnp.float32),
    scratch_types=[
        pltpu.VMEM((TPW, S * 128), jnp.int32),
        pltpu.VMEM((4, 128, D), jnp.float32),
        pltpu.VMEM((4, D // 8, 8, 129), jnp.float32),
        pltpu.SemaphoreType.DMA,
        pltpu.SemaphoreType.DMA,
    ],
)
def _gather_kernel(ids_hbm, table_hbm, out_hbm, ids_v, rows_v, stg_v,
                   gsem, ssem):
    wid = lax.axis_index("s") * NC + lax.axis_index("c")
    gtt0 = wid * TPW
    pltpu.sync_copy(ids_hbm.at[pl.ds(gtt0, TPW)], ids_v)
    iota = lax.iota(jnp.int32, 16)
    e4a = iota // 8
    e8v = iota - e4a * 8
    e4b = e4a + 2

    def start_gather(it, b):
        tl = it // S
        s = it - tl * S
        pltpu.async_copy(
            table_hbm.at[ids_v.at[tl, pl.ds(s * 128, 128)]],
            rows_v.at[b], gsem)

    def wait_gather(b):
        pltpu.make_async_copy(
            table_hbm.at[ids_v.at[0, pl.ds(0, 128)]], rows_v.at[b],
            gsem).wait()

    def start_store(it, b):
        tl = it // S
        s = it - tl * S
        pltpu.async_copy(stg_v.at[b, :, :, pl.ds(0, 128)],
                         out_hbm.at[s, :, gtt0 + tl], ssem)

    def wait_store(b):
        pltpu.make_async_copy(stg_v.at[b, :, :, pl.ds(0, 128)],
                              out_hbm.at[0, :, gtt0], ssem).wait()

    def transpose_block(b):
        rows = rows_v.at[b]
        stg = stg_v.at[b]
        for t in range(128):
            tv = jnp.full((16,), t, jnp.int32)
            v0 = rows[t, pl.ds(0, 16)]
            v1 = rows[t, pl.ds(16, 16)]
            plsc.store_scatter(stg, [e4a, e8v, tv], v0)
            plsc.store_scatter(stg, [e4b, e8v, tv], v1)

    for b in range(4):
        start_gather(b, b)

    def body(j, carry):
        for b in range(4):
            it = j * 4 + b
            wait_gather(b)

            @pl.when(it >= 4)
            def _():
                wait_store(b)

            transpose_block(b)
            start_store(it, b)

            @pl.when(it < NBLK - 4)
            def _():
                start_gather(it + 4, b)

        return carry

    lax.fori_loop(0, NBLK // 4, body, 0)
    for b in range(4):
        wait_store(b)


def kernel(token_ids, weight):
    ids2 = (token_ids.astype(jnp.int32).T
            .reshape(S, TT, 128).transpose(1, 0, 2).reshape(TT, S * 128))
    out5 = _gather_kernel(ids2, weight)
    return out5.transpose(2, 4, 0, 1, 3).reshape(T, S, D)
